# baseline (speedup 1.0000x reference)
import functools

import jax
import jax.numpy as jnp
from jax import lax
from jax.experimental import pallas as pl
from jax.experimental.pallas import tpu as pltpu
from jax.experimental.pallas import tpu_sc as plsc

N = 10000
E = 100000
IN = 128
OUT = 128
R = 3
NC = 2
NT = 16
NW = NC * NT
GROUP = 64
NBUF = 4
CH = 8
NCHUNK = 19
NGRP = CH * NCHUNK
EPAD = NW * NGRP * GROUP
ROWS_PAD = 10112
ROWS_PER_TILE = ROWS_PAD // NT

_BM = 1000


def _tc_matmul_body(x_ref, w_ref, y_ref):
    y_ref[...] = jnp.dot(x_ref[...], w_ref[0], preferred_element_type=jnp.float32)


def _tc_matmul(x, weight):
    return pl.pallas_call(
        _tc_matmul_body,
        grid=(R, N // _BM),
        in_specs=[
            pl.BlockSpec((_BM, IN), lambda r, i: (i, 0)),
            pl.BlockSpec((1, IN, OUT), lambda r, i: (r, 0, 0)),
        ],
        out_specs=pl.BlockSpec((_BM, OUT), lambda r, i: (r * (N // _BM) + i, 0)),
        out_shape=jax.ShapeDtypeStruct((R * N, OUT), jnp.float32),
    )(x, weight)


def _sc_scatter_body(ytab_hbm, isrc_hbm, idst_hbm, out_hbm,
                     s0_v, s1_v, d0_v, d1_v, b0, b1, b2, b3, agg_sp,
                     g0, g1, g2, g3, t0, t1, t2, t3, si0, si1, di0, di1):
    c = lax.axis_index("c")
    s = lax.axis_index("s")
    w = c * NT + s
    sbuf = (s0_v, s1_v)
    dbuf = (d0_v, d1_v)
    bufs = (b0, b1, b2, b3)
    gsem = (g0, g1, g2, g3)
    ssem = (t0, t1, t2, t3)
    isem = (si0, si1)
    dsem = (di0, di1)

    def load_idx_chunk(k, ring, wait):
        src_hbm = isrc_hbm.at[w, pl.ds(k * CH, CH)]
        dst_hbm = idst_hbm.at[w, pl.ds(k * CH, CH)]
        if wait:
            pltpu.sync_copy(src_hbm, sbuf[ring])
            pltpu.sync_copy(dst_hbm, dbuf[ring])
        else:
            pltpu.async_copy(src_hbm, sbuf[ring], isem[ring])
            pltpu.async_copy(dst_hbm, dbuf[ring], dsem[ring])

    def wait_idx_chunk(ring):
        pltpu.make_async_copy(isrc_hbm.at[w, pl.ds(0, CH)], sbuf[ring], isem[ring]).wait()
        pltpu.make_async_copy(idst_hbm.at[w, pl.ds(0, CH)], dbuf[ring], dsem[ring]).wait()

    def fire_gather(b, ring, grp):
        pltpu.async_copy(ytab_hbm.at[sbuf[ring].at[grp]], bufs[b], gsem[b])

    def wait_gather(b):
        pltpu.make_async_copy(ytab_hbm.at[sbuf[0].at[0]], bufs[b], gsem[b]).wait()

    def fire_scatter(b, ring, grp):
        pltpu.async_copy(bufs[b], agg_sp.at[dbuf[ring].at[grp]], ssem[b], add=True)

    def wait_scatter(b):
        pltpu.make_async_copy(bufs[b], agg_sp.at[dbuf[0].at[0]], ssem[b]).wait()

    zval = jnp.zeros((16,), jnp.float32)

    def zero_row(i, carry):
        for j in range(OUT // 16):
            b0[i, pl.ds(j * 16, 16)] = zval
        return carry

    lax.fori_loop(0, GROUP, zero_row, 0)
    row0 = s * ROWS_PER_TILE
    for k in range(ROWS_PER_TILE // GROUP):
        pltpu.sync_copy(b0, agg_sp.at[pl.ds(row0 + k * GROUP, GROUP)])
    rem = ROWS_PER_TILE % GROUP
    if rem:
        pltpu.sync_copy(b0.at[pl.ds(0, rem)],
                        agg_sp.at[pl.ds(row0 + (ROWS_PER_TILE // GROUP) * GROUP, rem)])
    plsc.subcore_barrier()

    def step(j, ring_p, ring_q, peel_first, maybe_last):
        b = j % NBUF
        wait_gather(b)
        fire_scatter(b, ring_p, j)
        b2 = (j + 2) % NBUF
        if not (peel_first and j < 2):
            wait_scatter(b2)
        if j < CH - 2:
            fire_gather(b2, ring_p, j + 2)
        elif not maybe_last:
            fire_gather(b2, ring_q, j - (CH - 2))

    def half_steps(k, ring_p, ring_q, peel_first):
        step(0, ring_p, ring_q, peel_first, False)
        step(1, ring_p, ring_q, peel_first, False)

        @pl.when(k + 1 < NCHUNK)
        def _():
            load_idx_chunk(k + 1, ring_q, wait=False)
        for j in range(2, CH - 2):
            step(j, ring_p, ring_q, peel_first, False)

        @pl.when(k + 1 < NCHUNK)
        def _():
            wait_idx_chunk(ring_q)
        for j in range(CH - 2, CH):
            step(j, ring_p, ring_q, peel_first, True)

            @pl.when(k + 1 < NCHUNK)
            def _():
                fire_gather((j + 2) % NBUF, ring_q, j - (CH - 2))

    load_idx_chunk(0, 0, wait=True)
    fire_gather(0, 0, 0)
    fire_gather(1, 0, 1)
    half_steps(0, 0, 1, peel_first=True)

    def body(k, carry):
        p = lax.rem(k, 2)

        @pl.when(p == 0)
        def _():
            half_steps(k, 0, 1, peel_first=False)

        @pl.when(p == 1)
        def _():
            half_steps(k, 1, 0, peel_first=False)
        return carry

    lax.fori_loop(1, NCHUNK, body, 0)
    wait_scatter((CH - 2) % NBUF)
    wait_scatter((CH - 1) % NBUF)
    plsc.subcore_barrier()
    pltpu.sync_copy(agg_sp.at[pl.ds(row0, ROWS_PER_TILE)],
                    out_hbm.at[c, pl.ds(row0, ROWS_PER_TILE)])


_sc_scatter = functools.partial(
    pl.kernel,
    out_type=jax.ShapeDtypeStruct((NC, ROWS_PAD, OUT), jnp.float32),
    mesh=plsc.VectorSubcoreMesh(core_axis_name="c", subcore_axis_name="s"),
    scratch_types=[
        pltpu.VMEM((CH, GROUP), jnp.int32),
        pltpu.VMEM((CH, GROUP), jnp.int32),
        pltpu.VMEM((CH, GROUP), jnp.int32),
        pltpu.VMEM((CH, GROUP), jnp.int32),
        pltpu.VMEM((GROUP, OUT), jnp.float32),
        pltpu.VMEM((GROUP, OUT), jnp.float32),
        pltpu.VMEM((GROUP, OUT), jnp.float32),
        pltpu.VMEM((GROUP, OUT), jnp.float32),
        pltpu.VMEM_SHARED((ROWS_PAD, OUT), jnp.float32),
    ] + [pltpu.SemaphoreType.DMA] * 12,
)(_sc_scatter_body)


def kernel(x, edge_index_r0, edge_index_r1, edge_index_r2, weight, h_bias):
    ytab = _tc_matmul(x, weight)

    gidx = jnp.concatenate([
        edge_index_r0[0],
        edge_index_r1[0] + N,
        edge_index_r2[0] + 2 * N,
    ])
    dst = jnp.concatenate([edge_index_r0[1], edge_index_r1[1], edge_index_r2[1]])
    pad = EPAD - R * E
    pad_iota = jax.lax.iota(jnp.int32, pad)
    gidx = jnp.concatenate([gidx, pad_iota % (R * N)])
    dst = jnp.concatenate([dst, N + pad_iota % (ROWS_PAD - N)])
    isrc = gidx.reshape(NW, NGRP, GROUP)
    idst = dst.reshape(NW, NGRP, GROUP)

    agg = _sc_scatter(ytab, isrc, idst)
    return agg[0, :N] + agg[1, :N] + h_bias


# speedup vs baseline: 1.0553x; 1.0553x over previous
import functools

import jax
import jax.numpy as jnp
from jax import lax
from jax.experimental import pallas as pl
from jax.experimental.pallas import tpu as pltpu
from jax.experimental.pallas import tpu_sc as plsc

N = 10000
E = 100000
IN = 128
OUT = 128
R = 3
NC = 2
NT = 16
NW = NC * NT
GROUP = 64
NBUF = 4
CH = 8
NCHUNK = 19
NGRP = CH * NCHUNK
EPAD = NW * NGRP * GROUP
ROWS_PAD = 10112
ROWS_PER_TILE = ROWS_PAD // NT

_BM = 1000


def _tc_matmul_body(x_ref, w_ref, y_ref):
    y_ref[...] = jnp.dot(x_ref[...], w_ref[0], preferred_element_type=jnp.float32)


def _tc_matmul(x, weight):
    return pl.pallas_call(
        _tc_matmul_body,
        grid=(R, N // _BM),
        in_specs=[
            pl.BlockSpec((_BM, IN), lambda r, i: (i, 0)),
            pl.BlockSpec((1, IN, OUT), lambda r, i: (r, 0, 0)),
        ],
        out_specs=pl.BlockSpec((_BM, OUT), lambda r, i: (r * (N // _BM) + i, 0)),
        out_shape=jax.ShapeDtypeStruct((R * N, OUT), jnp.float32),
    )(x, weight)


def _sc_scatter_body(ytab_hbm, eidx_hbm, out_hbm,
                     s0_v, s1_v, d0_v, d1_v, b0, b1, b2, b3, agg_sp,
                     g0, g1, g2, g3, si0, si1, di0, di1):
    c = lax.axis_index("c")
    s = lax.axis_index("s")
    w = c * NT + s
    sbuf = (s0_v, s1_v)
    dbuf = (d0_v, d1_v)
    bufs = (b0, b1, b2, b3)
    gsem = (g0, g1, g2, g3)
    isem = (si0, si1)
    dsem = (di0, di1)

    def load_idx_chunk(k, ring, wait):
        src_hbm = eidx_hbm.at[0, w, pl.ds(k * CH, CH)]
        dst_hbm = eidx_hbm.at[1, w, pl.ds(k * CH, CH)]
        if wait:
            pltpu.sync_copy(src_hbm, sbuf[ring])
            pltpu.sync_copy(dst_hbm, dbuf[ring])
        else:
            pltpu.async_copy(src_hbm, sbuf[ring], isem[ring])
            pltpu.async_copy(dst_hbm, dbuf[ring], dsem[ring])

    def wait_idx_chunk(ring):
        pltpu.make_async_copy(eidx_hbm.at[0, w, pl.ds(0, CH)], sbuf[ring], isem[ring]).wait()
        pltpu.make_async_copy(eidx_hbm.at[1, w, pl.ds(0, CH)], dbuf[ring], dsem[ring]).wait()

    def fire_gather(b, ring, grp):
        pltpu.async_copy(ytab_hbm.at[sbuf[ring].at[grp]], bufs[b], gsem[b])

    def wait_gather(b):
        pltpu.make_async_copy(ytab_hbm.at[sbuf[0].at[0]], bufs[b], gsem[b]).wait()

    zval = jnp.zeros((16,), jnp.float32)

    def zero_row(i, carry):
        for j in range(OUT // 16):
            b0[i, pl.ds(j * 16, 16)] = zval
        return carry

    lax.fori_loop(0, GROUP, zero_row, 0)
    row0 = s * ROWS_PER_TILE
    for k in range(ROWS_PER_TILE // GROUP):
        pltpu.sync_copy(b0, agg_sp.at[pl.ds(row0 + k * GROUP, GROUP)])
    rem = ROWS_PER_TILE % GROUP
    if rem:
        pltpu.sync_copy(b0.at[pl.ds(0, rem)],
                        agg_sp.at[pl.ds(row0 + (ROWS_PER_TILE // GROUP) * GROUP, rem)])
    plsc.subcore_barrier()

    load_idx_chunk(0, 0, wait=True)
    for b in range(NBUF):
        fire_gather(b, 0, b)
    load_idx_chunk(1, 1, wait=False)

    def body(k, carry):
        p = lax.rem(k, 2)
        q = lax.rem(k + 1, 2)

        def on_ring(ring_p, ring_q):
            @pl.when(k + 1 < NCHUNK)
            def _():
                wait_idx_chunk(ring_q)
            for b in range(NBUF):
                wait_gather(b)
                pltpu.sync_copy(bufs[b], agg_sp.at[dbuf[ring_p].at[b]], add=True)
                fire_gather(b, ring_p, NBUF + b)
            for b in range(NBUF):
                wait_gather(b)
                pltpu.sync_copy(bufs[b], agg_sp.at[dbuf[ring_p].at[NBUF + b]], add=True)

                @pl.when(k + 1 < NCHUNK)
                def _():
                    fire_gather(b, ring_q, b)

            @pl.when(k + 2 < NCHUNK)
            def _():
                load_idx_chunk(k + 2, ring_p, wait=False)

        @pl.when(p == 0)
        def _():
            on_ring(0, 1)

        @pl.when(p == 1)
        def _():
            on_ring(1, 0)
        return carry

    lax.fori_loop(0, NCHUNK, body, 0)
    plsc.subcore_barrier()
    pltpu.sync_copy(agg_sp.at[pl.ds(row0, ROWS_PER_TILE)],
                    out_hbm.at[c, pl.ds(row0, ROWS_PER_TILE)])


_sc_scatter = functools.partial(
    pl.kernel,
    out_type=jax.ShapeDtypeStruct((NC, ROWS_PAD, OUT), jnp.float32),
    mesh=plsc.VectorSubcoreMesh(core_axis_name="c", subcore_axis_name="s"),
    scratch_types=[
        pltpu.VMEM((CH, GROUP), jnp.int32),
        pltpu.VMEM((CH, GROUP), jnp.int32),
        pltpu.VMEM((CH, GROUP), jnp.int32),
        pltpu.VMEM((CH, GROUP), jnp.int32),
        pltpu.VMEM((GROUP, OUT), jnp.float32),
        pltpu.VMEM((GROUP, OUT), jnp.float32),
        pltpu.VMEM((GROUP, OUT), jnp.float32),
        pltpu.VMEM((GROUP, OUT), jnp.float32),
        pltpu.VMEM_SHARED((ROWS_PAD, OUT), jnp.float32),
        pltpu.SemaphoreType.DMA,
        pltpu.SemaphoreType.DMA,
        pltpu.SemaphoreType.DMA,
        pltpu.SemaphoreType.DMA,
        pltpu.SemaphoreType.DMA,
        pltpu.SemaphoreType.DMA,
        pltpu.SemaphoreType.DMA,
        pltpu.SemaphoreType.DMA,
    ],
)(_sc_scatter_body)


def kernel(x, edge_index_r0, edge_index_r1, edge_index_r2, weight, h_bias):
    ytab = _tc_matmul(x, weight)

    pad = EPAD - R * E
    pad_iota = jax.lax.iota(jnp.int32, pad)
    off1 = jnp.array([[N], [0]], jnp.int32)
    eidx = jnp.concatenate([
        edge_index_r0,
        edge_index_r1 + off1,
        edge_index_r2 + 2 * off1,
        jnp.stack([pad_iota % (R * N), N + pad_iota % (ROWS_PAD - N)]),
    ], axis=1).reshape(2, NW, NGRP, GROUP)

    agg = _sc_scatter(ytab, eidx)
    return agg[0, :N] + agg[1, :N] + h_bias
